# baseline (speedup 1.0000x reference)
import functools

import jax
import jax.numpy as jnp
from jax import lax
from jax.experimental import pallas as pl
from jax.experimental.pallas import tpu as pltpu
from jax.experimental.pallas import tpu_sc as plsc

N_NODES = 10000
N_EDGES = 320000
D_FEAT = 128

N_PAD = 10240
B_EDGES = 125
NUM_CORES = 2
NUM_SUBCORES = 16
NUM_TILES = NUM_CORES * NUM_SUBCORES
BLKS_PER_TILE = N_EDGES // (B_EDGES * NUM_TILES)
CHUNK_BLKS = BLKS_PER_TILE // 2
ROWS_PER_TILE = N_PAD // NUM_SUBCORES
ZROWS = 128


def _sc_gather_scatter(x, src3, dst3):
    mesh = plsc.VectorSubcoreMesh(core_axis_name="c", subcore_axis_name="s")

    @functools.partial(
        pl.kernel,
        out_type=jax.ShapeDtypeStruct((NUM_CORES, N_PAD, D_FEAT), jnp.float32),
        mesh=mesh,
        scratch_types=[
            pltpu.VMEM((CHUNK_BLKS, B_EDGES), jnp.int32),
            pltpu.VMEM((CHUNK_BLKS, B_EDGES), jnp.int32),
            pltpu.VMEM((ZROWS, D_FEAT), jnp.float32),
            pltpu.VMEM((ZROWS, D_FEAT), jnp.float32),
            pltpu.VMEM_SHARED((N_PAD, D_FEAT), jnp.float32),
            pltpu.SemaphoreType.DMA,
            pltpu.SemaphoreType.DMA,
            pltpu.SemaphoreType.DMA,
            pltpu.SemaphoreType.DMA,
        ],
    )
    def k(x_hbm, src_hbm, dst_hbm, out_hbm, src_v, dst_v, rows_a, rows_b, acc,
          gsem_a, gsem_b, ssem_a, ssem_b):
        cid = lax.axis_index("c")
        sid = lax.axis_index("s")
        wid = cid * NUM_SUBCORES + sid

        zero = jnp.zeros((16,), jnp.float32)

        @pl.loop(0, ZROWS)
        def _(r):
            @pl.loop(0, D_FEAT // 16)
            def _(c):
                rows_a.at[r, pl.ds(c * 16, 16)][...] = zero

        @pl.loop(0, ROWS_PER_TILE // ZROWS)
        def _(z):
            pltpu.sync_copy(
                rows_a,
                acc.at[pl.ds(sid * ROWS_PER_TILE + z * ZROWS, ZROWS)])

        plsc.subcore_barrier()

        @pl.loop(0, BLKS_PER_TILE // CHUNK_BLKS)
        def _(ch):
            pltpu.sync_copy(src_hbm.at[wid, pl.ds(ch * CHUNK_BLKS, CHUNK_BLKS)],
                            src_v)
            pltpu.sync_copy(dst_hbm.at[wid, pl.ds(ch * CHUNK_BLKS, CHUNK_BLKS)],
                            dst_v)

            @pl.loop(0, CHUNK_BLKS)
            def _(i):
                ra = rows_a.at[pl.ds(0, B_EDGES)]
                pltpu.sync_copy(x_hbm.at[src_v.at[i]], ra)

        plsc.subcore_barrier()

        pltpu.sync_copy(
            acc.at[pl.ds(sid * ROWS_PER_TILE, ROWS_PER_TILE)],
            out_hbm.at[cid, pl.ds(sid * ROWS_PER_TILE, ROWS_PER_TILE)])

    return k(x, src3, dst3)


def _tc_combine(partial):
    def body(p_ref, o_ref):
        o_ref[...] = p_ref[0] + p_ref[1]

    nb = 10
    return pl.pallas_call(
        body,
        out_shape=jax.ShapeDtypeStruct((N_NODES, D_FEAT), jnp.float32),
        grid=(nb,),
        in_specs=[pl.BlockSpec((NUM_CORES, N_NODES // nb, D_FEAT),
                               lambda i: (0, i, 0))],
        out_specs=pl.BlockSpec((N_NODES // nb, D_FEAT), lambda i: (i, 0)),
    )(partial)


def kernel(x, edge_index):
    src3 = edge_index[0].reshape(NUM_TILES, BLKS_PER_TILE, B_EDGES)
    dst3 = edge_index[1].reshape(NUM_TILES, BLKS_PER_TILE, B_EDGES)
    partial = _sc_gather_scatter(x, src3, dst3)
    return _tc_combine(partial)


# speedup vs baseline: 1.2470x; 1.2470x over previous
import functools

import jax
import jax.numpy as jnp
from jax import lax
from jax.experimental import pallas as pl
from jax.experimental.pallas import tpu as pltpu
from jax.experimental.pallas import tpu_sc as plsc

N_NODES = 10000
N_EDGES = 320000
D_FEAT = 128

N_PAD = 10240
B_EDGES = 100
NUM_CORES = 2
NUM_SUBCORES = 16
NUM_TILES = NUM_CORES * NUM_SUBCORES
BLKS_PER_TILE = N_EDGES // (B_EDGES * NUM_TILES)
CHUNK_BLKS = 25
NBUF = 3
ROWS_PER_TILE = N_PAD // NUM_SUBCORES


def _sc_gather_scatter(x, edge3):
    mesh = plsc.VectorSubcoreMesh(core_axis_name="c", subcore_axis_name="s")

    @functools.partial(
        pl.kernel,
        out_type=jax.ShapeDtypeStruct((NUM_CORES, N_PAD, D_FEAT), jnp.float32),
        mesh=mesh,
        scratch_types=[
            pltpu.VMEM((CHUNK_BLKS, B_EDGES), jnp.int32),
            pltpu.VMEM((CHUNK_BLKS, B_EDGES), jnp.int32),
            pltpu.VMEM((B_EDGES, D_FEAT), jnp.float32),
            pltpu.VMEM((B_EDGES, D_FEAT), jnp.float32),
            pltpu.VMEM((B_EDGES, D_FEAT), jnp.float32),
            pltpu.VMEM_SHARED((N_PAD, D_FEAT), jnp.float32),
            pltpu.SemaphoreType.DMA,
            pltpu.SemaphoreType.DMA,
            pltpu.SemaphoreType.DMA,
            pltpu.SemaphoreType.DMA,
            pltpu.SemaphoreType.DMA,
            pltpu.SemaphoreType.DMA,
        ],
    )
    def k(x_hbm, e_hbm, out_hbm, src_v, dst_v, r0, r1, r2, acc,
          g0, g1, g2, s0, s1, s2):
        bufs = [r0, r1, r2]
        gsems = [g0, g1, g2]
        ssems = [s0, s1, s2]

        cid = lax.axis_index("c")
        sid = lax.axis_index("s")
        wid = cid * NUM_SUBCORES + sid

        zero = jnp.zeros((16,), jnp.float32)

        @pl.loop(0, B_EDGES * (D_FEAT // 16))
        def _(i):
            r = i // (D_FEAT // 16)
            c = i % (D_FEAT // 16)
            r0.at[r, pl.ds(c * 16, 16)][...] = zero

        @pl.loop(0, ROWS_PER_TILE // B_EDGES)
        def _(z):
            pltpu.sync_copy(
                r0, acc.at[pl.ds(sid * ROWS_PER_TILE + z * B_EDGES, B_EDGES)])

        pltpu.sync_copy(
            r0.at[pl.ds(0, ROWS_PER_TILE % B_EDGES)],
            acc.at[pl.ds(sid * ROWS_PER_TILE
                         + (ROWS_PER_TILE // B_EDGES) * B_EDGES,
                         ROWS_PER_TILE % B_EDGES)])

        plsc.subcore_barrier()

        @pl.loop(0, BLKS_PER_TILE // CHUNK_BLKS)
        def _(ch):
            pltpu.sync_copy(e_hbm.at[0, wid, ch], src_v)
            pltpu.sync_copy(e_hbm.at[1, wid, ch], dst_v)

            gd = [None] * CHUNK_BLKS
            sd = [None] * CHUNK_BLKS
            for j in range(CHUNK_BLKS):
                b = j % NBUF
                if j >= NBUF:
                    sd[j - NBUF].wait()
                gd[j] = pltpu.async_copy(
                    x_hbm.at[src_v.at[j]], bufs[b], gsems[b])
                if j >= 2:
                    gd[j - 2].wait()
                    sd[j - 2] = pltpu.async_copy(
                        bufs[(j - 2) % NBUF], acc.at[dst_v.at[j - 2]],
                        ssems[(j - 2) % NBUF], add=True)
            for j in range(CHUNK_BLKS - 2, CHUNK_BLKS):
                gd[j].wait()
                sd[j] = pltpu.async_copy(
                    bufs[j % NBUF], acc.at[dst_v.at[j]],
                    ssems[j % NBUF], add=True)
            for j in range(CHUNK_BLKS - NBUF, CHUNK_BLKS):
                sd[j].wait()

        plsc.subcore_barrier()

        pltpu.sync_copy(
            acc.at[pl.ds(sid * ROWS_PER_TILE, ROWS_PER_TILE)],
            out_hbm.at[cid, pl.ds(sid * ROWS_PER_TILE, ROWS_PER_TILE)])

    return k(x, edge3)


def _tc_combine(partial):
    def body(p_ref, o_ref):
        o_ref[...] = p_ref[0] + p_ref[1]

    nb = 10
    return pl.pallas_call(
        body,
        out_shape=jax.ShapeDtypeStruct((N_NODES, D_FEAT), jnp.float32),
        grid=(nb,),
        in_specs=[pl.BlockSpec((NUM_CORES, N_NODES // nb, D_FEAT),
                               lambda i: (0, i, 0))],
        out_specs=pl.BlockSpec((N_NODES // nb, D_FEAT), lambda i: (i, 0)),
    )(partial)


def kernel(x, edge_index):
    edge5 = edge_index.reshape(2, NUM_TILES, BLKS_PER_TILE // CHUNK_BLKS,
                               CHUNK_BLKS, B_EDGES)
    partial = _sc_gather_scatter(x, edge5)
    return _tc_combine(partial)


# speedup vs baseline: 1.2699x; 1.0184x over previous
import functools

import jax
import jax.numpy as jnp
from jax import lax
from jax.experimental import pallas as pl
from jax.experimental.pallas import tpu as pltpu
from jax.experimental.pallas import tpu_sc as plsc

N_NODES = 10000
N_EDGES = 320000
D_FEAT = 128

N_PAD = 10240
B_EDGES = 80
NUM_CORES = 2
NUM_SUBCORES = 16
NUM_TILES = NUM_CORES * NUM_SUBCORES
EDGES_PER_TILE = N_EDGES // NUM_TILES
BLKS_PER_TILE = EDGES_PER_TILE // B_EDGES
CHUNK_BLKS = 25
CHUNK_EDGES = CHUNK_BLKS * B_EDGES
NBUF = 4
GDEPTH = 3
ROWS_PER_TILE = N_PAD // NUM_SUBCORES
VECS_PER_ROW = B_EDGES // 16


def _sc_gather_scatter(x, e1):
    mesh = plsc.VectorSubcoreMesh(core_axis_name="c", subcore_axis_name="s")

    @functools.partial(
        pl.kernel,
        out_type=jax.ShapeDtypeStruct((NUM_CORES, N_PAD, D_FEAT), jnp.float32),
        mesh=mesh,
        scratch_types=[
            pltpu.VMEM((CHUNK_EDGES,), jnp.int32),
            pltpu.VMEM((CHUNK_EDGES,), jnp.int32),
            pltpu.VMEM((CHUNK_BLKS, B_EDGES), jnp.int32),
            pltpu.VMEM((B_EDGES, D_FEAT), jnp.float32),
            pltpu.VMEM((B_EDGES, D_FEAT), jnp.float32),
            pltpu.VMEM((B_EDGES, D_FEAT), jnp.float32),
            pltpu.VMEM((B_EDGES, D_FEAT), jnp.float32),
            pltpu.VMEM_SHARED((N_PAD, D_FEAT), jnp.float32),
            pltpu.SemaphoreType.DMA,
            pltpu.SemaphoreType.DMA,
            pltpu.SemaphoreType.DMA,
            pltpu.SemaphoreType.DMA,
            pltpu.SemaphoreType.DMA,
            pltpu.SemaphoreType.DMA,
            pltpu.SemaphoreType.DMA,
            pltpu.SemaphoreType.DMA,
        ],
    )
    def k(x_hbm, e_hbm, out_hbm, src_v, dst_v, dst2, r0, r1, r2, r3, acc,
          g0, g1, g2, g3, s0, s1, s2, s3):
        bufs = [r0, r1, r2, r3]
        gsems = [g0, g1, g2, g3]
        ssems = [s0, s1, s2, s3]

        cid = lax.axis_index("c")
        sid = lax.axis_index("s")
        wid = cid * NUM_SUBCORES + sid

        zero = jnp.zeros((16,), jnp.float32)

        @pl.loop(0, B_EDGES * (D_FEAT // 16))
        def _(i):
            r = i // (D_FEAT // 16)
            c = i % (D_FEAT // 16)
            r0.at[r, pl.ds(c * 16, 16)][...] = zero

        @pl.loop(0, ROWS_PER_TILE // B_EDGES)
        def _(z):
            pltpu.sync_copy(
                r0, acc.at[pl.ds(sid * ROWS_PER_TILE + z * B_EDGES, B_EDGES)])

        plsc.subcore_barrier()

        @pl.loop(0, BLKS_PER_TILE // CHUNK_BLKS)
        def _(ch):
            base = wid * EDGES_PER_TILE + ch * CHUNK_EDGES
            pltpu.sync_copy(e_hbm.at[pl.ds(base, CHUNK_EDGES)], src_v)
            pltpu.sync_copy(e_hbm.at[pl.ds(N_EDGES + base, CHUNK_EDGES)],
                            dst_v)

            @pl.loop(0, CHUNK_EDGES // 16)
            def _(v):
                r = v // VECS_PER_ROW
                c = v % VECS_PER_ROW
                dst2.at[r, pl.ds(c * 16, 16)][...] = \
                    dst_v.at[pl.ds(v * 16, 16)][...]

            gd = [None] * CHUNK_BLKS
            sd = [None] * CHUNK_BLKS

            def scat(j):
                return pltpu.async_copy(
                    bufs[j % NBUF], acc.at[dst2.at[j]], ssems[j % NBUF],
                    add=True)

            for j in range(CHUNK_BLKS):
                b = j % NBUF
                if j >= NBUF:
                    sd[j - NBUF].wait()
                gd[j] = pltpu.async_copy(
                    x_hbm.at[src_v.at[pl.ds(j * B_EDGES, B_EDGES)]],
                    bufs[b], gsems[b])
                if j >= GDEPTH:
                    gd[j - GDEPTH].wait()
                    sd[j - GDEPTH] = scat(j - GDEPTH)
            for j in range(CHUNK_BLKS - GDEPTH, CHUNK_BLKS):
                gd[j].wait()
                sd[j] = scat(j)
            for j in range(CHUNK_BLKS - NBUF, CHUNK_BLKS):
                sd[j].wait()

        plsc.subcore_barrier()

        pltpu.sync_copy(
            acc.at[pl.ds(sid * ROWS_PER_TILE, ROWS_PER_TILE)],
            out_hbm.at[cid, pl.ds(sid * ROWS_PER_TILE, ROWS_PER_TILE)])

    return k(x, e1)


def _tc_combine(partial):
    def body(p_ref, o_ref):
        o_ref[...] = p_ref[0] + p_ref[1]

    nb = 5
    return pl.pallas_call(
        body,
        out_shape=jax.ShapeDtypeStruct((N_NODES, D_FEAT), jnp.float32),
        grid=(nb,),
        in_specs=[pl.BlockSpec((NUM_CORES, N_NODES // nb, D_FEAT),
                               lambda i: (0, i, 0))],
        out_specs=pl.BlockSpec((N_NODES // nb, D_FEAT), lambda i: (i, 0)),
    )(partial)


def kernel(x, edge_index):
    e1 = edge_index.reshape(2 * N_EDGES)
    partial = _sc_gather_scatter(x, e1)
    return _tc_combine(partial)


# speedup vs baseline: 1.3221x; 1.0411x over previous
import functools

import jax
import jax.numpy as jnp
from jax import lax
from jax.experimental import pallas as pl
from jax.experimental.pallas import tpu as pltpu
from jax.experimental.pallas import tpu_sc as plsc

N_NODES = 10000
N_EDGES = 320000
D_FEAT = 128

N_PAD = 10240
STRIPE = 128
N_STRIPES = N_EDGES // STRIPE
NUM_CORES = 2
NUM_SUBCORES = 16
NUM_TILES = NUM_CORES * NUM_SUBCORES
STRIPES_PER_TILE = N_STRIPES // NUM_TILES
N_EXTRA = N_STRIPES % NUM_TILES
B_EDGES = 64
BLKS_PER_TILE = STRIPES_PER_TILE * STRIPE // B_EDGES
CHUNK_BLKS = 52
CHUNK_EDGES = CHUNK_BLKS * B_EDGES
NBUF = 4
GDEPTH = 3
ROWS_PER_TILE = N_PAD // NUM_SUBCORES
VECS_PER_ROW = B_EDGES // 16


def _sc_gather_scatter(x, edge_index):
    mesh = plsc.VectorSubcoreMesh(core_axis_name="c", subcore_axis_name="s")

    @functools.partial(
        pl.kernel,
        out_type=jax.ShapeDtypeStruct((NUM_CORES, N_PAD, D_FEAT), jnp.float32),
        mesh=mesh,
        scratch_types=[
            pltpu.VMEM((CHUNK_EDGES,), jnp.int32),
            pltpu.VMEM((CHUNK_EDGES,), jnp.int32),
            pltpu.VMEM((CHUNK_BLKS, B_EDGES), jnp.int32),
            pltpu.VMEM((B_EDGES, D_FEAT), jnp.float32),
            pltpu.VMEM((B_EDGES, D_FEAT), jnp.float32),
            pltpu.VMEM((B_EDGES, D_FEAT), jnp.float32),
            pltpu.VMEM((B_EDGES, D_FEAT), jnp.float32),
            pltpu.VMEM_SHARED((N_PAD, D_FEAT), jnp.float32),
            pltpu.SemaphoreType.DMA,
            pltpu.SemaphoreType.DMA,
            pltpu.SemaphoreType.DMA,
            pltpu.SemaphoreType.DMA,
            pltpu.SemaphoreType.DMA,
            pltpu.SemaphoreType.DMA,
            pltpu.SemaphoreType.DMA,
            pltpu.SemaphoreType.DMA,
        ],
    )
    def k(x_hbm, e_hbm, out_hbm, src_v, dst_v, dst2, r0, r1, r2, r3, acc,
          g0, g1, g2, g3, s0, s1, s2, s3):
        bufs = [r0, r1, r2, r3]
        gsems = [g0, g1, g2, g3]
        ssems = [s0, s1, s2, s3]

        cid = lax.axis_index("c")
        sid = lax.axis_index("s")
        wid = cid * NUM_SUBCORES + sid
        base_edge = wid * STRIPES_PER_TILE * STRIPE

        zero = jnp.zeros((16,), jnp.float32)

        @pl.loop(0, B_EDGES * (D_FEAT // 16))
        def _(i):
            r = i // (D_FEAT // 16)
            c = i % (D_FEAT // 16)
            r0.at[r, pl.ds(c * 16, 16)][...] = zero

        @pl.loop(0, ROWS_PER_TILE // B_EDGES)
        def _(z):
            pltpu.sync_copy(
                r0, acc.at[pl.ds(sid * ROWS_PER_TILE + z * B_EDGES, B_EDGES)])

        plsc.subcore_barrier()

        @pl.loop(0, BLKS_PER_TILE // CHUNK_BLKS)
        def _(ch):
            cb = base_edge + ch * CHUNK_EDGES
            pltpu.sync_copy(e_hbm.at[0, pl.ds(cb, CHUNK_EDGES)], src_v)
            pltpu.sync_copy(e_hbm.at[1, pl.ds(cb, CHUNK_EDGES)], dst_v)

            @pl.loop(0, CHUNK_EDGES // 16)
            def _(v):
                r = v // VECS_PER_ROW
                c = v % VECS_PER_ROW
                dst2.at[r, pl.ds(c * 16, 16)][...] = \
                    dst_v.at[pl.ds(v * 16, 16)][...]

            gd = [None] * CHUNK_BLKS
            sd = [None] * CHUNK_BLKS

            def scat(j):
                return pltpu.async_copy(
                    bufs[j % NBUF], acc.at[dst2.at[j]], ssems[j % NBUF],
                    add=True)

            for j in range(CHUNK_BLKS):
                b = j % NBUF
                if j >= NBUF:
                    sd[j - NBUF].wait()
                gd[j] = pltpu.async_copy(
                    x_hbm.at[src_v.at[pl.ds(j * B_EDGES, B_EDGES)]],
                    bufs[b], gsems[b])
                if j >= GDEPTH:
                    gd[j - GDEPTH].wait()
                    sd[j - GDEPTH] = scat(j - GDEPTH)
            for j in range(CHUNK_BLKS - GDEPTH, CHUNK_BLKS):
                gd[j].wait()
                sd[j] = scat(j)
            for j in range(CHUNK_BLKS - NBUF, CHUNK_BLKS):
                sd[j].wait()

        @pl.when(wid < N_EXTRA)
        def _():
            tb = (N_STRIPES - N_EXTRA) * STRIPE + wid * STRIPE
            pltpu.sync_copy(e_hbm.at[0, pl.ds(tb, STRIPE)],
                            src_v.at[pl.ds(0, STRIPE)])
            pltpu.sync_copy(e_hbm.at[1, pl.ds(tb, STRIPE)],
                            dst_v.at[pl.ds(0, STRIPE)])

            @pl.loop(0, STRIPE // 16)
            def _(v):
                r = v // VECS_PER_ROW
                c = v % VECS_PER_ROW
                dst2.at[r, pl.ds(c * 16, 16)][...] = \
                    dst_v.at[pl.ds(v * 16, 16)][...]

            for t in range(STRIPE // B_EDGES):
                pltpu.sync_copy(
                    x_hbm.at[src_v.at[pl.ds(t * B_EDGES, B_EDGES)]], r0)
                pltpu.sync_copy(r0, acc.at[dst2.at[t]], add=True)

        plsc.subcore_barrier()

        pltpu.sync_copy(
            acc.at[pl.ds(sid * ROWS_PER_TILE, ROWS_PER_TILE)],
            out_hbm.at[cid, pl.ds(sid * ROWS_PER_TILE, ROWS_PER_TILE)])

    return k(x, edge_index)


def _tc_combine(partial):
    def body(p_ref, o_ref):
        o_ref[...] = p_ref[0] + p_ref[1]

    nb = 5
    return pl.pallas_call(
        body,
        out_shape=jax.ShapeDtypeStruct((N_NODES, D_FEAT), jnp.float32),
        grid=(nb,),
        in_specs=[pl.BlockSpec((NUM_CORES, N_NODES // nb, D_FEAT),
                               lambda i: (0, i, 0))],
        out_specs=pl.BlockSpec((N_NODES // nb, D_FEAT), lambda i: (i, 0)),
    )(partial)


def kernel(x, edge_index):
    partial = _sc_gather_scatter(x, edge_index)
    return _tc_combine(partial)


# speedup vs baseline: 1.3392x; 1.0129x over previous
import functools

import jax
import jax.numpy as jnp
from jax import lax
from jax.experimental import pallas as pl
from jax.experimental.pallas import tpu as pltpu
from jax.experimental.pallas import tpu_sc as plsc

N_NODES = 10000
N_EDGES = 320000
D_FEAT = 128

N_PAD = 10240
STRIPE = 128
N_STRIPES = N_EDGES // STRIPE
NUM_CORES = 2
NUM_SUBCORES = 16
NUM_TILES = NUM_CORES * NUM_SUBCORES
STRIPES_PER_TILE = N_STRIPES // NUM_TILES
N_EXTRA = N_STRIPES % NUM_TILES
B_EDGES = 64
BLKS_PER_TILE = STRIPES_PER_TILE * STRIPE // B_EDGES
CHUNK_BLKS = 52
CHUNK_EDGES = CHUNK_BLKS * B_EDGES
NBUF = 4
GDEPTH = 3
ROWS_PER_TILE = N_PAD // NUM_SUBCORES
VECS_PER_ROW = B_EDGES // 16


def _sc_gather_scatter(x, edge_index):
    mesh = plsc.VectorSubcoreMesh(core_axis_name="c", subcore_axis_name="s")

    @functools.partial(
        pl.kernel,
        out_type=jax.ShapeDtypeStruct((NUM_CORES, N_PAD, D_FEAT), jnp.float32),
        mesh=mesh,
        scratch_types=[
            pltpu.VMEM((CHUNK_EDGES,), jnp.int32),
            pltpu.VMEM((CHUNK_EDGES,), jnp.int32),
            pltpu.VMEM((CHUNK_BLKS, B_EDGES), jnp.int32),
            pltpu.VMEM((B_EDGES, D_FEAT), jnp.float32),
            pltpu.VMEM((B_EDGES, D_FEAT), jnp.float32),
            pltpu.VMEM((B_EDGES, D_FEAT), jnp.float32),
            pltpu.VMEM((B_EDGES, D_FEAT), jnp.float32),
            pltpu.VMEM_SHARED((N_PAD, D_FEAT), jnp.float32),
            pltpu.SemaphoreType.DMA,
            pltpu.SemaphoreType.DMA,
            pltpu.SemaphoreType.DMA,
            pltpu.SemaphoreType.DMA,
            pltpu.SemaphoreType.DMA,
            pltpu.SemaphoreType.DMA,
            pltpu.SemaphoreType.DMA,
            pltpu.SemaphoreType.DMA,
        ],
    )
    def k(x_hbm, e_hbm, out_hbm, src_v, dst_v, dst2, r0, r1, r2, r3, acc,
          g0, g1, g2, g3, s0, s1, s2, s3):
        bufs = [r0, r1, r2, r3]
        gsems = [g0, g1, g2, g3]
        ssems = [s0, s1, s2, s3]

        cid = lax.axis_index("c")
        sid = lax.axis_index("s")
        wid = cid * NUM_SUBCORES + sid
        base_edge = wid * STRIPES_PER_TILE * STRIPE

        zero = jnp.zeros((16,), jnp.float32)

        @pl.loop(0, B_EDGES * (D_FEAT // 16))
        def _(i):
            r = i // (D_FEAT // 16)
            c = i % (D_FEAT // 16)
            r0.at[r, pl.ds(c * 16, 16)][...] = zero

        @pl.loop(0, ROWS_PER_TILE // B_EDGES)
        def _(z):
            pltpu.sync_copy(
                r0, acc.at[pl.ds(sid * ROWS_PER_TILE + z * B_EDGES, B_EDGES)])

        plsc.subcore_barrier()

        @pl.loop(0, BLKS_PER_TILE // CHUNK_BLKS)
        def _(ch):
            cb = base_edge + ch * CHUNK_EDGES
            pltpu.sync_copy(e_hbm.at[0, pl.ds(cb, CHUNK_EDGES)], src_v)
            pltpu.sync_copy(e_hbm.at[1, pl.ds(cb, CHUNK_EDGES)], dst_v)

            @pl.loop(0, CHUNK_EDGES // 16)
            def _(v):
                r = v // VECS_PER_ROW
                c = v % VECS_PER_ROW
                dst2.at[r, pl.ds(c * 16, 16)][...] = \
                    dst_v.at[pl.ds(v * 16, 16)][...]

            gd = [None] * CHUNK_BLKS
            sd = [None] * CHUNK_BLKS

            def scat(j):
                return pltpu.async_copy(
                    bufs[j % NBUF], acc.at[dst2.at[j]], ssems[j % NBUF],
                    add=True)

            for j in range(CHUNK_BLKS):
                b = j % NBUF
                if j >= NBUF:
                    sd[j - NBUF].wait()
                gd[j] = pltpu.async_copy(
                    x_hbm.at[src_v.at[pl.ds(j * B_EDGES, B_EDGES)]],
                    bufs[b], gsems[b])
                if j >= GDEPTH:
                    gd[j - GDEPTH].wait()
                    sd[j - GDEPTH] = scat(j - GDEPTH)
            for j in range(CHUNK_BLKS - GDEPTH, CHUNK_BLKS):
                gd[j].wait()
                sd[j] = scat(j)
            for j in range(CHUNK_BLKS - NBUF, CHUNK_BLKS):
                sd[j].wait()

        @pl.when(wid < N_EXTRA)
        def _():
            tb = (N_STRIPES - N_EXTRA) * STRIPE + wid * STRIPE
            pltpu.sync_copy(e_hbm.at[0, pl.ds(tb, STRIPE)],
                            src_v.at[pl.ds(0, STRIPE)])
            pltpu.sync_copy(e_hbm.at[1, pl.ds(tb, STRIPE)],
                            dst_v.at[pl.ds(0, STRIPE)])

            @pl.loop(0, STRIPE // 16)
            def _(v):
                r = v // VECS_PER_ROW
                c = v % VECS_PER_ROW
                dst2.at[r, pl.ds(c * 16, 16)][...] = \
                    dst_v.at[pl.ds(v * 16, 16)][...]

            for t in range(STRIPE // B_EDGES):
                pltpu.sync_copy(
                    x_hbm.at[src_v.at[pl.ds(t * B_EDGES, B_EDGES)]], r0)
                pltpu.sync_copy(r0, acc.at[dst2.at[t]], add=True)

        plsc.subcore_barrier()

        pltpu.sync_copy(
            acc.at[pl.ds(sid * ROWS_PER_TILE, ROWS_PER_TILE)],
            out_hbm.at[cid, pl.ds(sid * ROWS_PER_TILE, ROWS_PER_TILE)])

    return k(x, edge_index)


def _tc_combine(partial):
    def body(p_ref, o_ref):
        o_ref[...] = p_ref[0] + p_ref[1]

    nb = 2
    return pl.pallas_call(
        body,
        out_shape=jax.ShapeDtypeStruct((N_NODES, D_FEAT), jnp.float32),
        grid=(nb,),
        in_specs=[pl.BlockSpec((NUM_CORES, N_NODES // nb, D_FEAT),
                               lambda i: (0, i, 0))],
        out_specs=pl.BlockSpec((N_NODES // nb, D_FEAT), lambda i: (i, 0)),
    )(partial)


def kernel(x, edge_index):
    partial = _sc_gather_scatter(x, edge_index)
    return _tc_combine(partial)
